# Initial kernel scaffold; baseline (speedup 1.0000x reference)
import jax
import jax.numpy as jnp
from jax.experimental import pallas as pl


def kernel(x_t, batch, ln_gamma, ln_beta, W, b):
    raise NotImplementedError("write your pallas kernel here")


# speedup vs baseline: 1.4937x; 1.4937x over previous
import functools
import jax
import jax.numpy as jnp
from jax import lax
from jax.experimental import pallas as pl
from jax.experimental.pallas import tpu as pltpu
from jax.experimental.pallas import tpu_sc as plsc

N = 1600000
S = 4096
D = 3
EPS = 1e-5

NC = 2
NS = 16
NW = NC * NS
ROWS_PER_W = N // NW
BLK_ROWS = 2000
NBLK = ROWS_PER_W // BLK_ROWS
GRP = BLK_ROWS // 16

@functools.cache
def _build_phase1():
    mesh = plsc.VectorSubcoreMesh(
        core_axis_name="c", subcore_axis_name="s", num_cores=NC, num_subcores=NS
    )
    return functools.partial(
        pl.kernel,
        mesh=mesh,
        compiler_params=pltpu.CompilerParams(needs_layout_passes=False),
        out_type=[
            jax.ShapeDtypeStruct((NW, D * S), jnp.float32),
            jax.ShapeDtypeStruct((NW, S), jnp.int32),
        ],
        scratch_types=[
            pltpu.VMEM((BLK_ROWS * D,), jnp.float32),
            pltpu.VMEM((BLK_ROWS,), jnp.int32),
            pltpu.VMEM((D * S,), jnp.float32),
            pltpu.VMEM((S,), jnp.int32),
        ],
    )(_sc_phase1_body)


def _sc_phase1_body(x_hbm, b_hbm, outx_hbm, outc_hbm, xv, bv, accx, accc):
    wid = lax.axis_index("s") * NC + lax.axis_index("c")
    x_base = wid * (ROWS_PER_W * D)
    b_base = wid * ROWS_PER_W

    zf = jnp.zeros((16,), jnp.float32)
    zi = jnp.zeros((16,), jnp.int32)

    def zero_f(i, c):
        accx[pl.ds(i * 16, 16)] = zf
        return c

    def zero_i(i, c):
        accc[pl.ds(i * 16, 16)] = zi
        return c

    lax.fori_loop(0, (D * S) // 16, zero_f, 0)
    lax.fori_loop(0, S // 16, zero_i, 0)

    ii = lax.iota(jnp.int32, 16)
    rows = []
    cbase = []
    for m in range(D):
        p = ii + 16 * m
        r = p // D
        rows.append(r)
        cbase.append((p - r * D) * S)
    ones_i = jnp.ones((16,), jnp.int32)

    def grp_body(g, c):
        g16 = g * 16
        g48 = g * 48
        bt = bv[pl.ds(g16, 16)]
        plsc.addupdate_scatter(accc, [bt], ones_i)
        for m in range(D):
            xm = xv[pl.ds(g48 + 16 * m, 16)]
            seg = plsc.load_gather(bv, [g16 + rows[m]])
            plsc.addupdate_scatter(accx, [cbase[m] + seg], xm)
        return c

    def blk_body(k, c):
        pltpu.sync_copy(x_hbm.at[pl.ds(x_base + k * (BLK_ROWS * D), BLK_ROWS * D)], xv)
        pltpu.sync_copy(b_hbm.at[pl.ds(b_base + k * BLK_ROWS, BLK_ROWS)], bv)
        lax.fori_loop(0, GRP, grp_body, 0)
        return c

    lax.fori_loop(0, NBLK, blk_body, 0)

    pltpu.sync_copy(accx, outx_hbm.at[wid])
    pltpu.sync_copy(accc, outc_hbm.at[wid])


def _tc_phase2_body(ps_ref, pc_ref, prm_ref, pred_ref, lig_ref):
    s = jnp.sum(ps_ref[...], axis=0, keepdims=True)
    cnt = jnp.sum(pc_ref[...], axis=0, keepdims=True)
    cf = jnp.maximum(cnt.astype(jnp.float32), 1.0)
    mx = s[:, 0:S] / cf
    my = s[:, S:2 * S] / cf
    mz = s[:, 2 * S:3 * S] / cf
    mu = (mx + my + mz) * (1.0 / 3.0)
    dx = mx - mu
    dy = my - mu
    dz = mz - mu
    var = (dx * dx + dy * dy + dz * dz) * (1.0 / 3.0)
    rs = lax.rsqrt(var + EPS)
    g0 = prm_ref[0]
    g1 = prm_ref[1]
    g2 = prm_ref[2]
    be0 = prm_ref[3]
    be1 = prm_ref[4]
    be2 = prm_ref[5]
    w0 = prm_ref[6]
    w1 = prm_ref[7]
    w2 = prm_ref[8]
    bb = prm_ref[9]
    x0 = dx * rs * g0 + be0
    x1 = dy * rs * g1 + be1
    x2 = dz * rs * g2 + be2
    pred_ref[...] = x0 * w0 + x1 * w1 + x2 * w2 + bb
    lig_ref[...] = cnt


@jax.jit
def kernel(x_t, batch, ln_gamma, ln_beta, W, b):
    x_flat = x_t.reshape(-1)
    sums_p, cnt_p = _build_phase1()(x_flat, batch)
    params = jnp.concatenate(
        [ln_gamma, ln_beta, W.reshape(-1), b, jnp.zeros((6,), jnp.float32)]
    )
    pred, lig = pl.pallas_call(
        _tc_phase2_body,
        out_shape=[
            jax.ShapeDtypeStruct((1, S), jnp.float32),
            jax.ShapeDtypeStruct((1, S), jnp.int32),
        ],
        in_specs=[
            pl.BlockSpec(memory_space=pltpu.VMEM),
            pl.BlockSpec(memory_space=pltpu.VMEM),
            pl.BlockSpec(memory_space=pltpu.SMEM),
        ],
    )(sums_p, cnt_p, params)
    return pred.reshape(S, 1), lig.reshape(S)


# speedup vs baseline: 100.1624x; 67.0571x over previous
import functools
import jax
import jax.numpy as jnp
from jax import lax
from jax.experimental import pallas as pl
from jax.experimental.pallas import tpu as pltpu
from jax.experimental.pallas import tpu_sc as plsc

N = 1600000
S = 4096
D = 3
EPS = 1e-5

NC = 2
NS = 16
NW = NC * NS
BLK_ROWS = 2048
TOT_BLKS = -(-N // BLK_ROWS)
LAST_BLK = TOT_BLKS - 1
LAST_ROWS = N - LAST_BLK * BLK_ROWS

@functools.cache
def _build_phase1():
    mesh = plsc.VectorSubcoreMesh(
        core_axis_name="c", subcore_axis_name="s", num_cores=NC, num_subcores=NS
    )
    return functools.partial(
        pl.kernel,
        mesh=mesh,
        compiler_params=pltpu.CompilerParams(needs_layout_passes=False),
        out_type=[
            jax.ShapeDtypeStruct((NW, D * S), jnp.float32),
            jax.ShapeDtypeStruct((NW, S), jnp.int32),
        ],
        scratch_types=[
            pltpu.VMEM((D, BLK_ROWS), jnp.float32),
            pltpu.VMEM((BLK_ROWS,), jnp.int32),
            pltpu.VMEM((D, BLK_ROWS), jnp.float32),
            pltpu.VMEM((BLK_ROWS,), jnp.int32),
            pltpu.VMEM((D * S,), jnp.float32),
            pltpu.VMEM((S,), jnp.int32),
            pltpu.SemaphoreType.DMA,
            pltpu.SemaphoreType.DMA,
        ],
    )(_sc_phase1_body)


MAIN_BLKS = 24
EXTRA_W = TOT_BLKS - NW * MAIN_BLKS


def _sc_phase1_body(
    x_hbm, b_hbm, outx_hbm, outc_hbm, xv0, bv0, xv1, bv1, accx, accc, sem0, sem1
):
    wid = lax.axis_index("s") * NC + lax.axis_index("c")

    zf = jnp.zeros((16,), jnp.float32)
    zi = jnp.zeros((16,), jnp.int32)

    def zero_f(i, c):
        accx[pl.ds(i * 16, 16)] = zf
        return c

    def zero_i(i, c):
        accc[pl.ds(i * 16, 16)] = zi
        return c

    lax.fori_loop(0, (D * S) // 16, zero_f, 0)
    lax.fori_loop(0, S // 16, zero_i, 0)

    ii = lax.iota(jnp.int32, 16)
    lane0 = ii == 0
    lane15 = ii == 15
    ip1 = jnp.minimum(ii + 1, 15)
    im1 = jnp.maximum(ii - 1, 0)
    ones_i = jnp.ones((16,), jnp.int32)
    zf16 = jnp.zeros((16,), jnp.float32)
    zi16 = jnp.zeros((16,), jnp.int32)

    def start_blk(bufx, bufb, sem, bid):
        off = bid * BLK_ROWS
        pltpu.make_async_copy(x_hbm.at[:, pl.ds(off, BLK_ROWS)], bufx, sem).start()
        pltpu.make_async_copy(b_hbm.at[pl.ds(off, BLK_ROWS)], bufb, sem).start()

    def wait_blk(bufx, bufb, sem):
        pltpu.make_async_copy(x_hbm.at[:, pl.ds(0, BLK_ROWS)], bufx, sem).wait()
        pltpu.make_async_copy(b_hbm.at[pl.ds(0, BLK_ROWS)], bufb, sem).wait()

    def flush(open_s, ax, ay, az, ac):
        openv = jnp.full((16,), open_s, jnp.int32)
        plsc.addupdate_scatter(accc, [openv], ac)
        for m, a in enumerate((ax, ay, az)):
            plsc.addupdate_scatter(accx, [openv + m * S], a)

    def grp_slow(bt, new_open, xs, carry):
        flush(*carry)
        nxt = jnp.take_along_axis(bt, ip1, axis=0)
        prv = jnp.take_along_axis(bt, im1, axis=0)
        boundary = (bt != nxt) | lane15
        startm = (bt != prv) | lane0
        s = plsc.cummax(jnp.where(startm, ii, 0))
        plsc.addupdate_scatter(accc, [bt], ii - s + 1, mask=boundary)
        for m in range(D):
            cs = plsc.cumsum(xs[m])
            ecs = cs - xs[m]
            pcs = jnp.take_along_axis(ecs, s, axis=0)
            plsc.addupdate_scatter(accx, [bt + m * S], cs - pcs, mask=boundary)
        return (new_open, zf16, zf16, zf16, zi16)

    UNROLL = 4

    def make_grpu(bufx, bufb):
        def grp16(g16, carry):
            xs = [bufx[m, pl.ds(g16, 16)] for m in range(D)]
            bt = bufb[pl.ds(g16, 16)]
            b0 = bt[0]
            b15 = bt[15]
            same = (b0 == b15) & (b0 == carry[0])

            def fast(ops):
                op, fax, fay, faz, fac = ops
                return (op, fax + xs[0], fay + xs[1], faz + xs[2], fac + ones_i)

            def slow(ops):
                return grp_slow(bt, b15, xs, ops)

            return lax.cond(same, fast, slow, carry)

        def grpu(g, carry):
            base = g * (16 * UNROLL)
            bfirst = bufb[pl.ds(base, 16)]
            blast = bufb[pl.ds(base + 16 * (UNROLL - 1), 16)]
            span_same = (bfirst[0] == blast[15]) & (bfirst[0] == carry[0])

            def fastspan(ops):
                op, fax, fay, faz, fac = ops
                for u in range(UNROLL):
                    xs = [bufx[m, pl.ds(base + u * 16, 16)] for m in range(D)]
                    fax = fax + xs[0]
                    fay = fay + xs[1]
                    faz = faz + xs[2]
                return (op, fax, fay, faz, fac + UNROLL * ones_i)

            def slowspan(ops):
                for u in range(UNROLL):
                    ops = grp16(base + u * 16, ops)
                return ops

            return lax.cond(span_same, fastspan, slowspan, carry)

        return grpu

    def process(bufx, bufb, ngrp):
        first = bufb[pl.ds(0, 16)]
        carry = lax.fori_loop(
            0,
            ngrp // UNROLL,
            make_grpu(bufx, bufb),
            (first[0], zf16, zf16, zf16, zi16),
        )
        flush(*carry)

    start_blk(xv0, bv0, sem0, wid)
    start_blk(xv1, bv1, sem1, wid + NW)

    def pipe(t, c):
        wait_blk(xv0, bv0, sem0)
        process(xv0, bv0, BLK_ROWS // 16)
        start_blk(xv0, bv0, sem0, wid + NW * (2 * t + 2))
        wait_blk(xv1, bv1, sem1)
        process(xv1, bv1, BLK_ROWS // 16)
        start_blk(xv1, bv1, sem1, wid + NW * (2 * t + 3))
        return c

    lax.fori_loop(0, MAIN_BLKS // 2 - 1, pipe, 0)
    wait_blk(xv0, bv0, sem0)
    process(xv0, bv0, BLK_ROWS // 16)
    wait_blk(xv1, bv1, sem1)
    process(xv1, bv1, BLK_ROWS // 16)

    xoff = NW * MAIN_BLKS * BLK_ROWS

    @pl.when(wid < EXTRA_W - 1)
    def _extra_full():
        off = xoff + wid * BLK_ROWS
        pltpu.sync_copy(x_hbm.at[:, pl.ds(off, BLK_ROWS)], xv0)
        pltpu.sync_copy(b_hbm.at[pl.ds(off, BLK_ROWS)], bv0)
        process(xv0, bv0, BLK_ROWS // 16)

    @pl.when(wid == EXTRA_W - 1)
    def _extra_tail():
        off = xoff + (EXTRA_W - 1) * BLK_ROWS
        pltpu.sync_copy(
            x_hbm.at[:, pl.ds(off, LAST_ROWS)], xv0.at[:, pl.ds(0, LAST_ROWS)]
        )
        pltpu.sync_copy(b_hbm.at[pl.ds(off, LAST_ROWS)], bv0.at[pl.ds(0, LAST_ROWS)])
        process(xv0, bv0, LAST_ROWS // 16)

    pltpu.sync_copy(accx, outx_hbm.at[wid])
    pltpu.sync_copy(accc, outc_hbm.at[wid])


def _tc_phase2_body(ps_ref, pc_ref, prm_ref, pred_ref, lig_ref):
    s = jnp.sum(ps_ref[...], axis=0, keepdims=True)
    cnt = jnp.sum(pc_ref[...], axis=0, keepdims=True)
    cf = jnp.maximum(cnt.astype(jnp.float32), 1.0)
    mx = s[:, 0:S] / cf
    my = s[:, S:2 * S] / cf
    mz = s[:, 2 * S:3 * S] / cf
    mu = (mx + my + mz) * (1.0 / 3.0)
    dx = mx - mu
    dy = my - mu
    dz = mz - mu
    var = (dx * dx + dy * dy + dz * dz) * (1.0 / 3.0)
    rs = lax.rsqrt(var + EPS)
    g0 = prm_ref[0]
    g1 = prm_ref[1]
    g2 = prm_ref[2]
    be0 = prm_ref[3]
    be1 = prm_ref[4]
    be2 = prm_ref[5]
    w0 = prm_ref[6]
    w1 = prm_ref[7]
    w2 = prm_ref[8]
    bb = prm_ref[9]
    x0 = dx * rs * g0 + be0
    x1 = dy * rs * g1 + be1
    x2 = dz * rs * g2 + be2
    pred_ref[...] = x0 * w0 + x1 * w1 + x2 * w2 + bb
    lig_ref[...] = cnt


@jax.jit
def kernel(x_t, batch, ln_gamma, ln_beta, W, b):
    x_cols = x_t.T
    sums_p, cnt_p = _build_phase1()(x_cols, batch)
    params = jnp.concatenate(
        [ln_gamma, ln_beta, W.reshape(-1), b, jnp.zeros((6,), jnp.float32)]
    )
    pred, lig = pl.pallas_call(
        _tc_phase2_body,
        out_shape=[
            jax.ShapeDtypeStruct((1, S), jnp.float32),
            jax.ShapeDtypeStruct((1, S), jnp.int32),
        ],
        in_specs=[
            pl.BlockSpec(memory_space=pltpu.VMEM),
            pl.BlockSpec(memory_space=pltpu.VMEM),
            pl.BlockSpec(memory_space=pltpu.SMEM),
        ],
    )(sums_p, cnt_p, params)
    return pred.reshape(S, 1), lig.reshape(S)
